# Initial kernel scaffold; baseline (speedup 1.0000x reference)
import jax
import jax.numpy as jnp
from jax.experimental import pallas as pl


def kernel(x):
    raise NotImplementedError("write your pallas kernel here")


# speedup vs baseline: 34.8282x; 34.8282x over previous
import functools

import jax
import jax.numpy as jnp
import numpy as np
from jax.experimental import pallas as pl
from jax.experimental.pallas import tpu as pltpu

_ALPHA = 0.98
_DELTA = 2.0
_R = 0.5
_S = 0.025
_EPS = 1e-6
_A = 1.0 - _S


def _pcen_kernel(x_ref, l_ref, apow_ref, o_ref, carry_ref, *, t_total, w):
    t = pl.program_id(1)
    lanes = jax.lax.broadcasted_iota(jnp.int32, x_ref.shape, 1)
    xb = jnp.where(lanes < (t_total - t * w), x_ref[...], 0.0)

    @pl.when(t == 0)
    def _():
        carry_ref[...] = jnp.broadcast_to(xb[:, 0:1], carry_ref.shape)

    sm = (
        jnp.dot(xb, l_ref[...], preferred_element_type=jnp.float32)
        + carry_ref[...] * apow_ref[...]
    )
    carry_ref[...] = jnp.broadcast_to(sm[:, w - 1 : w], carry_ref.shape)

    inv_pow = jnp.exp2(jnp.log2(sm + _EPS) * (-_ALPHA))
    o_ref[...] = jnp.sqrt(xb * inv_pow + _DELTA) - np.float32(np.sqrt(_DELTA))


def _build_consts(w):
    k = np.arange(w)[:, None].astype(np.float64)
    j = np.arange(w)[None, :].astype(np.float64)
    l_mat = np.where(j >= k, _S * _A ** (j - k), 0.0).astype(np.float32)
    apow = (_A ** (np.arange(w, dtype=np.float64) + 1.0)).astype(np.float32)
    return l_mat, apow.reshape(1, w)


@jax.jit
def kernel(x):
    b, c, t_total = x.shape
    rows = b * c
    x2 = x.reshape(rows, t_total)

    w = 256
    rt = 2048
    n_chunks = pl.cdiv(t_total, w)
    n_row_tiles = pl.cdiv(rows, rt)

    l_mat, apow = _build_consts(w)

    out = pl.pallas_call(
        functools.partial(_pcen_kernel, t_total=t_total, w=w),
        out_shape=jax.ShapeDtypeStruct((rows, t_total), jnp.float32),
        grid=(n_row_tiles, n_chunks),
        in_specs=[
            pl.BlockSpec((rt, w), lambda i, t: (i, t)),
            pl.BlockSpec((w, w), lambda i, t: (0, 0)),
            pl.BlockSpec((1, w), lambda i, t: (0, 0)),
        ],
        out_specs=pl.BlockSpec((rt, w), lambda i, t: (i, t)),
        scratch_shapes=[pltpu.VMEM((rt, w), jnp.float32)],
        compiler_params=pltpu.CompilerParams(
            dimension_semantics=("parallel", "arbitrary"),
        ),
        name="pcen",
    )(x2, jnp.asarray(l_mat), jnp.asarray(apow))

    return out.reshape(b, c, t_total)
